# Initial kernel scaffold; baseline (speedup 1.0000x reference)
import jax
import jax.numpy as jnp
from jax.experimental import pallas as pl


def kernel(x, edge_index, W1, b1, W2, b2):
    raise NotImplementedError("write your pallas kernel here")


# speedup vs baseline: 12.8437x; 12.8437x over previous
import functools

import jax
import jax.numpy as jnp
from jax import lax
from jax.experimental import pallas as pl
from jax.experimental.pallas import tpu as pltpu
from jax.experimental.pallas import tpu_sc as plsc

N = 10000
D = 128
E = 320000
NC, NS = 2, 16
NW = NC * NS
EPW = E // NW
K = 80
NCHUNK = EPW // K
NP = 10240
ROWS_PT = NP // NS
ZR = 128
DEG_N = 10240
DEG_PT = DEG_N // NS

_MESH = dict(core_axis_name="c", subcore_axis_name="s")


@functools.partial(
    pl.kernel,
    out_type=jax.ShapeDtypeStruct((NC, DEG_N), jnp.float32),
    mesh=plsc.VectorSubcoreMesh(**_MESH),
    scratch_types=[
        pltpu.VMEM((K,), jnp.int32),
        pltpu.VMEM((K,), jnp.float32),
        pltpu.VMEM((DEG_PT,), jnp.float32),
        pltpu.VMEM_SHARED((DEG_N,), jnp.float32),
    ],
)
def _deg_kernel(dst_hbm, out_hbm, dst_v, ones_v, zero_v, deg_sh):
    c = lax.axis_index("c")
    s = lax.axis_index("s")
    w = c * NS + s

    for i in range(K // 16):
        ones_v[pl.ds(i * 16, 16)] = jnp.ones((16,), jnp.float32)

    def _z(i, _):
        zero_v[pl.ds(i * 16, 16)] = jnp.zeros((16,), jnp.float32)
        return 0

    lax.fori_loop(0, DEG_PT // 16, _z, 0)
    pltpu.sync_copy(zero_v, deg_sh.at[pl.ds(s * DEG_PT, DEG_PT)])
    plsc.subcore_barrier()

    def _chunk(j, _):
        base = pl.multiple_of(w * EPW + j * K, 8)
        pltpu.sync_copy(dst_hbm.at[pl.ds(base, K)], dst_v)
        pltpu.sync_copy(ones_v, deg_sh.at[dst_v], add=True)
        return 0

    lax.fori_loop(0, NCHUNK, _chunk, 0)
    plsc.subcore_barrier()
    pltpu.sync_copy(
        deg_sh.at[pl.ds(s * DEG_PT, DEG_PT)],
        out_hbm.at[c, pl.ds(s * DEG_PT, DEG_PT)],
    )


@functools.partial(
    pl.kernel,
    out_type=jax.ShapeDtypeStruct((NC, NP, D), jnp.float32),
    mesh=plsc.VectorSubcoreMesh(**_MESH),
    scratch_types=[
        pltpu.VMEM((K,), jnp.int32),
        pltpu.VMEM((K,), jnp.int32),
        pltpu.VMEM((K, D), jnp.float32),
        pltpu.VMEM((ZR, D), jnp.float32),
        pltpu.VMEM_SHARED((NP, D), jnp.float32),
        pltpu.SemaphoreType.DMA,
    ],
)
def _agg_kernel(g_hbm, src_hbm, dst_hbm, out_hbm,
                src_v, dst_v, rows_v, zrows_v, agg_sh, sem):
    c = lax.axis_index("c")
    s = lax.axis_index("s")
    w = c * NS + s

    def _z(i, _):
        zrows_v[i // 8, pl.ds((i % 8) * 16, 16)] = jnp.zeros((16,), jnp.float32)
        return 0

    lax.fori_loop(0, ZR * 8, _z, 0)
    for r in range(ROWS_PT // ZR):
        pltpu.sync_copy(zrows_v, agg_sh.at[pl.ds(s * ROWS_PT + r * ZR, ZR)])
    plsc.subcore_barrier()

    def _chunk(j, _):
        base = pl.multiple_of(w * EPW + j * K, 8)
        pltpu.sync_copy(src_hbm.at[pl.ds(base, K)], src_v)
        pltpu.async_copy(g_hbm.at[src_v], rows_v, sem).wait()
        pltpu.sync_copy(dst_hbm.at[pl.ds(base, K)], dst_v)
        pltpu.sync_copy(rows_v, agg_sh.at[dst_v], add=True)
        return 0

    lax.fori_loop(0, NCHUNK, _chunk, 0)
    plsc.subcore_barrier()
    pltpu.sync_copy(
        agg_sh.at[pl.ds(s * ROWS_PT, ROWS_PT)],
        out_hbm.at[c, pl.ds(s * ROWS_PT, ROWS_PT)],
    )


_R = 1000


def _dinv(degA_ref, degB_ref):
    return lax.rsqrt(degA_ref[...] + degB_ref[...] + 1.0)


def _mm_scale_body(x_ref, w_ref, degA_ref, degB_ref, o_ref):
    h = jnp.dot(x_ref[...], w_ref[...], preferred_element_type=jnp.float32)
    o_ref[...] = h * _dinv(degA_ref, degB_ref)


def _layer2_body(g1_ref, aggA_ref, aggB_ref, degA_ref, degB_ref,
                 w2_ref, b1_ref, o_ref):
    dinv = _dinv(degA_ref, degB_ref)
    h = dinv * (aggA_ref[...] + aggB_ref[...] + g1_ref[...]) + b1_ref[...]
    h = jnp.maximum(h, 0.0)
    o_ref[...] = jnp.dot(h, w2_ref[...],
                         preferred_element_type=jnp.float32) * dinv


def _final_body(g2_ref, aggA_ref, aggB_ref, degA_ref, degB_ref,
                b2_ref, o_ref):
    dinv = _dinv(degA_ref, degB_ref)
    o_ref[...] = dinv * (aggA_ref[...] + aggB_ref[...] + g2_ref[...]) + b2_ref[...]


def _row_spec(width=D):
    return pl.BlockSpec((_R, width), lambda i: (i, 0))


def _full_spec(shape):
    return pl.BlockSpec(shape, lambda i: (0, 0))


def _mm_scale(x, W, degA, degB):
    return pl.pallas_call(
        _mm_scale_body,
        grid=(N // _R,),
        in_specs=[_row_spec(), _full_spec((D, D)), _row_spec(1), _row_spec(1)],
        out_specs=_row_spec(),
        out_shape=jax.ShapeDtypeStruct((N, D), jnp.float32),
    )(x, W, degA, degB)


def _layer2(g1, aggA, aggB, degA, degB, W2, b1):
    return pl.pallas_call(
        _layer2_body,
        grid=(N // _R,),
        in_specs=[_row_spec(), _row_spec(), _row_spec(), _row_spec(1),
                  _row_spec(1), _full_spec((D, D)), _full_spec((1, D))],
        out_specs=_row_spec(),
        out_shape=jax.ShapeDtypeStruct((N, D), jnp.float32),
    )(g1, aggA, aggB, degA, degB, W2, b1)


def _final(g2, aggA, aggB, degA, degB, b2):
    return pl.pallas_call(
        _final_body,
        grid=(N // _R,),
        in_specs=[_row_spec(), _row_spec(), _row_spec(), _row_spec(1),
                  _row_spec(1), _full_spec((1, D))],
        out_specs=_row_spec(),
        out_shape=jax.ShapeDtypeStruct((N, D), jnp.float32),
    )(g2, aggA, aggB, degA, degB, b2)


def kernel(x, edge_index, W1, b1, W2, b2):
    src = edge_index[0].astype(jnp.int32)
    dst = edge_index[1].astype(jnp.int32)
    b1r = b1.reshape(1, D)
    b2r = b2.reshape(1, D)

    deg_parts = _deg_kernel(dst)
    degA = deg_parts[0, :N].reshape(N, 1)
    degB = deg_parts[1, :N].reshape(N, 1)

    g1 = _mm_scale(x, W1, degA, degB)
    agg1 = _agg_kernel(g1, src, dst)
    g2 = _layer2(g1, agg1[0, :N], agg1[1, :N], degA, degB, W2, b1r)
    agg2 = _agg_kernel(g2, src, dst)
    return _final(g2, agg2[0, :N], agg2[1, :N], degA, degB, b2r)
